# baseline (speedup 1.0000x reference)
import functools

import jax
import jax.numpy as jnp
from jax import lax
from jax.experimental import pallas as pl
from jax.experimental.pallas import tpu as pltpu
from jax.experimental.pallas import tpu_sc as plsc

_B, _S, _D = 4, 8192, 768
_NC, _NS = 2, 16
_NW = _NC * _NS
_R = 16
_SPW = _S // _NW
_NBLK = _SPW // _R
_LANES = 16
_BLK = _R * _D


def _make_sc_add():
    mesh = plsc.VectorSubcoreMesh(core_axis_name="c", subcore_axis_name="s")
    f32 = jnp.float32
    buf = pltpu.VMEM((_BLK,), f32)
    dma = pltpu.SemaphoreType.DMA

    @functools.partial(
        pl.kernel,
        mesh=mesh,
        out_type=jax.ShapeDtypeStruct((_B * _S * _D,), f32),
        scratch_types=[buf] * 2 + [buf] * (_B * 2) + [dma] * 2 + [dma] * (_B * 2) + [dma] * (_B * 2),
    )
    def sc_add(x_hbm, pos_hbm, out_hbm, *scr):
        pos_v = scr[0:2]
        x_v = [scr[2 + 2 * b: 4 + 2 * b] for b in range(_B)]
        sp = scr[2 + 2 * _B: 4 + 2 * _B]
        sin = [scr[4 + 2 * _B + 2 * b: 6 + 2 * _B + 2 * b] for b in range(_B)]
        sout = [scr[4 + 4 * _B + 2 * b: 6 + 4 * _B + 2 * b] for b in range(_B)]

        wid = lax.axis_index("s") * _NC + lax.axis_index("c")
        base = wid * _SPW

        def pos_src(i):
            return pos_hbm.at[pl.ds((base + i * _R) * _D, _BLK)]

        def x_src(i, b):
            return x_hbm.at[pl.ds((b * _S + base + i * _R) * _D, _BLK)]

        def out_dst(i, b):
            return out_hbm.at[pl.ds((b * _S + base + i * _R) * _D, _BLK)]

        pltpu.async_copy(pos_src(0), pos_v[0], sp[0])
        for b in range(_B):
            pltpu.async_copy(x_src(0, b), x_v[b][0], sin[b][0])

        def step(t, carry):
            for p in range(2):
                q = 1 - p
                i = 2 * t + p

                @pl.when(i < _NBLK - 1)
                def _prefetch():
                    pltpu.async_copy(pos_src(i + 1), pos_v[q], sp[q])

                for b in range(_B):
                    @pl.when(i > 0)
                    def _drain_out():
                        pltpu.make_async_copy(x_v[b][q], out_dst(i - 1, b), sout[b][q]).wait()

                    @pl.when(i < _NBLK - 1)
                    def _next_in():
                        pltpu.async_copy(x_src(i + 1, b), x_v[b][q], sin[b][q])

                pltpu.make_async_copy(pos_src(i), pos_v[p], sp[p]).wait()
                for b in range(_B):
                    pltpu.make_async_copy(x_src(i, b), x_v[b][p], sin[b][p]).wait()

                    def add_row(j, c2):
                        row = j * _D
                        for col in range(_D // _LANES):
                            o = row + col * _LANES
                            plsc.addupdate(
                                x_v[b][p].at[pl.ds(o, _LANES)],
                                pos_v[p][pl.ds(o, _LANES)],
                            )
                        return c2

                    lax.fori_loop(0, _R, add_row, 0)
                    pltpu.async_copy(x_v[b][p], out_dst(i, b), sout[b][p])
            return carry

        lax.fori_loop(0, _NBLK // 2, step, 0)

        for b in range(_B):
            pltpu.make_async_copy(x_v[b][1], out_dst(_NBLK - 1, b), sout[b][1]).wait()

    return sc_add


_sc_add = _make_sc_add()


def kernel(x, pos_table):
    B, S, D = x.shape
    out = _sc_add(x.reshape(-1), pos_table[:S].reshape(-1))
    return out.reshape(B, S, D)


# speedup vs baseline: 3.0927x; 3.0927x over previous
import functools

import jax
import jax.numpy as jnp
from jax import lax
from jax.experimental import pallas as pl
from jax.experimental.pallas import tpu as pltpu
from jax.experimental.pallas import tpu_sc as plsc

_B, _S, _D = 4, 8192, 768
_NC, _NS = 2, 16
_NW = _NC * _NS
_R = 16
_SPW = _S // _NW
_NBLK = _SPW // _R
_LANES = 16
_BLK = _R * _D


def _make_sc_add():
    mesh = plsc.VectorSubcoreMesh(core_axis_name="c", subcore_axis_name="s")
    f32 = jnp.float32
    buf = pltpu.VMEM((_R, _D), f32)
    dma = pltpu.SemaphoreType.DMA

    @functools.partial(
        pl.kernel,
        mesh=mesh,
        out_type=jax.ShapeDtypeStruct((_B * _S, _D), f32),
        scratch_types=[buf] * 2 + [buf] * (_B * 2) + [dma] * 2 + [dma] * (_B * 2) + [dma] * (_B * 2),
    )
    def sc_add(x_hbm, pos_hbm, out_hbm, *scr):
        pos_v = scr[0:2]
        x_v = [scr[2 + 2 * b: 4 + 2 * b] for b in range(_B)]
        sp = scr[2 + 2 * _B: 4 + 2 * _B]
        sin = [scr[4 + 2 * _B + 2 * b: 6 + 2 * _B + 2 * b] for b in range(_B)]
        sout = [scr[4 + 4 * _B + 2 * b: 6 + 4 * _B + 2 * b] for b in range(_B)]

        wid = lax.axis_index("s") * _NC + lax.axis_index("c")
        base = wid * _SPW

        def pos_src(i):
            return pos_hbm.at[pl.ds(base + i * _R, _R), :]

        def x_src(i, b):
            return x_hbm.at[pl.ds(b * _S + base + i * _R, _R), :]

        def out_dst(i, b):
            return out_hbm.at[pl.ds(b * _S + base + i * _R, _R), :]

        pltpu.async_copy(pos_src(0), pos_v[0], sp[0])
        for b in range(_B):
            pltpu.async_copy(x_src(0, b), x_v[b][0], sin[b][0])

        def step(t, carry):
            for p in range(2):
                q = 1 - p
                i = 2 * t + p

                @pl.when(i < _NBLK - 1)
                def _prefetch():
                    pltpu.async_copy(pos_src(i + 1), pos_v[q], sp[q])

                for b in range(_B):
                    @pl.when(i > 0)
                    def _drain_out():
                        pltpu.make_async_copy(x_v[b][q], out_dst(i - 1, b), sout[b][q]).wait()

                    @pl.when(i < _NBLK - 1)
                    def _next_in():
                        pltpu.async_copy(x_src(i + 1, b), x_v[b][q], sin[b][q])

                pltpu.make_async_copy(pos_src(i), pos_v[p], sp[p]).wait()
                for b in range(_B):
                    pltpu.make_async_copy(x_src(i, b), x_v[b][p], sin[b][p]).wait()

                    def add_row(j, c2):
                        for col in range(_D // _LANES):
                            o = col * _LANES
                            plsc.addupdate(
                                x_v[b][p].at[j, pl.ds(o, _LANES)],
                                pos_v[p][j, pl.ds(o, _LANES)],
                            )
                        return c2

                    lax.fori_loop(0, _R, add_row, 0)
                    pltpu.async_copy(x_v[b][p], out_dst(i, b), sout[b][p])
            return carry

        lax.fori_loop(0, _NBLK // 2, step, 0)

        for b in range(_B):
            pltpu.make_async_copy(x_v[b][1], out_dst(_NBLK - 1, b), sout[b][1]).wait()

    return sc_add


_sc_add = _make_sc_add()


def kernel(x, pos_table):
    B, S, D = x.shape
    out = _sc_add(x.reshape(B * S, D), pos_table[:S])
    return out.reshape(B, S, D)
